# baseline (speedup 1.0000x reference)
import functools

import jax
import jax.numpy as jnp
from jax import lax
from jax.experimental import pallas as pl
from jax.experimental.pallas import tpu as pltpu
from jax.experimental.pallas import tpu_sc as plsc

NQ = 8
K = 1024
D = 256
N = 8192
TN = 1024

_NC = 2
_NS = 16
_NW = _NC * _NS


def _c2_body(cb_ref, c2_ref):
    c2_ref[...] = jnp.sum(cb_ref[...] * cb_ref[...], axis=2)


def _argmin_tile(r, cb, c2row):
    mm = jax.lax.dot_general(
        r, cb, (((1,), (1,)), ((), ())),
        preferred_element_type=jnp.float32)
    r2 = jnp.sum(r * r, axis=1, keepdims=True)
    d2 = (r2 - 2.0 * mm) + c2row[None, :]
    m = jnp.min(d2, axis=1, keepdims=True)
    iota = jax.lax.broadcasted_iota(jnp.int32, d2.shape, 1)
    return jnp.min(jnp.where(d2 == m, iota, K), axis=1)


def _stage0_body(x_ref, cb_ref, c2_ref, idx_ref):
    idx = _argmin_tile(x_ref[...], cb_ref[...], c2_ref[0])
    idx_ref[...] = idx.reshape(1, 1, TN)


def _stage_body(rin_ref, qprev_ref, cb_ref, c2_ref,
                rout_ref, idx_ref, loss_ref):
    i = pl.program_id(0)

    @pl.when(i == 0)
    def _init():
        loss_ref[...] = jnp.zeros_like(loss_ref)

    r = rin_ref[...] - qprev_ref[...]
    loss_ref[...] += jnp.sum(r * r).reshape(1, 1)
    idx = _argmin_tile(r, cb_ref[...], c2_ref[0])
    idx_ref[...] = idx.reshape(1, 1, TN)
    rout_ref[...] = r


def _epi_body(r7_ref, q0, q1, q2, q3, q4, q5, q6, q7,
              zq_ref, loss_ref):
    i = pl.program_id(0)

    @pl.when(i == 0)
    def _init():
        loss_ref[...] = jnp.zeros_like(loss_ref)

    zq = q0[...]
    for qref in (q1, q2, q3, q4, q5, q6, q7):
        zq = zq + qref[...]
    r8 = r7_ref[...] - q7[...]
    loss_ref[...] += jnp.sum(r8 * r8).reshape(1, 1)
    zq_ref[...] = zq


def _gather_body(bpw):
    def body(cb_hbm, idx_hbm, out_hbm, idx_v, rows_v, sem):
        wid = lax.axis_index("s") * _NC + lax.axis_index("c")
        base = wid * bpw
        pltpu.sync_copy(idx_hbm.at[pl.ds(base, bpw)], idx_v)
        pltpu.async_copy(cb_hbm.at[idx_v], rows_v, sem).wait()
        pltpu.sync_copy(rows_v, out_hbm.at[pl.ds(base, bpw)])
    return body


def _make_sc_gather(n_tok):
    bpw = n_tok // _NW
    return functools.partial(
        pl.kernel,
        mesh=plsc.VectorSubcoreMesh(core_axis_name="c", subcore_axis_name="s"),
        out_type=jax.ShapeDtypeStruct((n_tok, D), jnp.float32),
        scratch_types=[
            pltpu.VMEM((bpw,), jnp.int32),
            pltpu.VMEM((bpw, D), jnp.float32),
            pltpu.SemaphoreType.DMA,
        ],
    )(_gather_body(bpw))


def _tc_stage0(x, cb, c2q, n_tok):
    grid = (n_tok // TN,)
    return pl.pallas_call(
        _stage0_body,
        grid=grid,
        in_specs=[
            pl.BlockSpec((TN, D), lambda i: (i, 0)),
            pl.BlockSpec((K, D), lambda i: (0, 0)),
            pl.BlockSpec((1, K), lambda i: (0, 0)),
        ],
        out_specs=pl.BlockSpec((1, 1, TN), lambda i: (i, 0, 0)),
        out_shape=jax.ShapeDtypeStruct((n_tok // TN, 1, TN), jnp.int32),
        compiler_params=pltpu.CompilerParams(
            dimension_semantics=("arbitrary",)),
    )(x, cb, c2q)


def _tc_stage(r_in, q_prev, cb, c2q, n_tok):
    grid = (n_tok // TN,)
    return pl.pallas_call(
        _stage_body,
        grid=grid,
        in_specs=[
            pl.BlockSpec((TN, D), lambda i: (i, 0)),
            pl.BlockSpec((TN, D), lambda i: (i, 0)),
            pl.BlockSpec((K, D), lambda i: (0, 0)),
            pl.BlockSpec((1, K), lambda i: (0, 0)),
        ],
        out_specs=[
            pl.BlockSpec((TN, D), lambda i: (i, 0)),
            pl.BlockSpec((1, 1, TN), lambda i: (i, 0, 0)),
            pl.BlockSpec((1, 1), lambda i: (0, 0)),
        ],
        out_shape=[
            jax.ShapeDtypeStruct((n_tok, D), jnp.float32),
            jax.ShapeDtypeStruct((n_tok // TN, 1, TN), jnp.int32),
            jax.ShapeDtypeStruct((1, 1), jnp.float32),
        ],
        compiler_params=pltpu.CompilerParams(
            dimension_semantics=("arbitrary",)),
    )(r_in, q_prev, cb, c2q)


def _tc_epilogue(r7, quants, n_tok):
    grid = (n_tok // TN,)
    tile = pl.BlockSpec((TN, D), lambda i: (i, 0))
    return pl.pallas_call(
        _epi_body,
        grid=grid,
        in_specs=[tile] * 9,
        out_specs=[
            tile,
            pl.BlockSpec((1, 1), lambda i: (0, 0)),
        ],
        out_shape=[
            jax.ShapeDtypeStruct((n_tok, D), jnp.float32),
            jax.ShapeDtypeStruct((1, 1), jnp.float32),
        ],
        compiler_params=pltpu.CompilerParams(
            dimension_semantics=("arbitrary",)),
    )(r7, *quants)


def kernel(latent, codebooks):
    Bm, Tm, Dm = latent.shape
    n_tok = Bm * Tm
    x = latent.reshape(n_tok, Dm)

    c2 = pl.pallas_call(
        _c2_body,
        in_specs=[pl.BlockSpec((NQ, K, D), lambda: (0, 0, 0))],
        out_specs=pl.BlockSpec((NQ, K), lambda: (0, 0)),
        out_shape=jax.ShapeDtypeStruct((NQ, K), jnp.float32),
    )(codebooks)

    sc_gather = _make_sc_gather(n_tok)

    idx_list = []
    quants = []
    losses = []
    r = x
    for q in range(NQ):
        cb = codebooks[q]
        c2q = c2[q:q + 1]
        if q == 0:
            idx_t = _tc_stage0(x, cb, c2q, n_tok)
        else:
            r, idx_t, lpart = _tc_stage(r, quants[q - 1], cb, c2q, n_tok)
            losses.append(lpart)
        idx_flat = idx_t.reshape(n_tok)
        idx_list.append(idx_flat)
        quants.append(sc_gather(cb, idx_flat))

    zq, l7 = _tc_epilogue(r, quants, n_tok)
    losses.append(l7)

    z_q = zq.reshape(Bm, Tm, Dm)
    codes = jnp.stack(idx_list, axis=-1).reshape(Bm, Tm, NQ)
    q_loss = sum(jnp.squeeze(l) for l in losses) / (n_tok * Dm)
    return z_q, codes, q_loss
